# baseline (speedup 1.0000x reference)
import functools

import jax
import jax.numpy as jnp
from jax import lax
from jax.experimental import pallas as pl
from jax.experimental.pallas import tpu as pltpu
from jax.experimental.pallas import tpu_sc as plsc

B = 16384
V1 = 100001
EMB = 32
HID = 256
OUT = 128
NUM = 3
N_FEAT = 131

_TBLK = 512
_NTB = 196
_VPAD = _TBLK * _NTB

_NC, _NS = 2, 16
_NW = _NC * _NS
_BPW = B // _NW
_BPC = 128


_Q = _VPAD // 4


def _transpose_body(*refs):
    xs, outs = refs[:16], refs[16:]
    for t in range(4):
        o = outs[t]
        for a in range(4):
            o[:, a * EMB:(a + 1) * EMB] = xs[4 * t + a][...].T


def _tquarter_spec(a):
    return pl.BlockSpec((EMB, 4 * EMB), lambda i, a=a: (0, a * _NTB + i))


_tc_transpose = pl.pallas_call(
    _transpose_body,
    grid=(_NTB,),
    in_specs=[_tquarter_spec(a) for _ in range(4) for a in range(4)],
    out_specs=[pl.BlockSpec((4 * EMB, 4 * EMB), lambda i: (i, 0))
               for _ in range(4)],
    out_shape=[jax.ShapeDtypeStruct((_Q, 4 * EMB), jnp.float32)] * 4,
)


def _sc_gather_body(i0, i1, i2, i3, t0, t1, t2, t3, o0, o1, o2, o3,
                    idx0, idx1, idx2, idx3, r0, r1, r2, r3, sem):
    wid = lax.axis_index("s") * _NC + lax.axis_index("c")
    base = wid * _BPW
    idxs = (idx0, idx1, idx2, idx3)
    rows = (r0, r1, r2, r3)
    tabs = (t0, t1, t2, t3)
    encs = (i0, i1, i2, i3)
    outs = (o0, o1, o2, o3)
    for t in range(4):
        pltpu.sync_copy(encs[t].at[pl.ds(base, _BPW)], idxs[t])
    for c in range(_BPW // _BPC):
        cbase = c * _BPC
        cps = [pltpu.async_copy(tabs[t].at[idxs[t].at[pl.ds(cbase, _BPC)]],
                                rows[t], sem)
               for t in range(4)]
        for cp in cps:
            cp.wait()
        for t in range(4):
            pltpu.sync_copy(rows[t],
                            outs[t].at[pl.ds(base + cbase, _BPC), :])


@functools.cache
def _sc_gather():
    return pl.kernel(
        _sc_gather_body,
        out_type=[jax.ShapeDtypeStruct((B, 4 * EMB), jnp.float32)] * 4,
        mesh=plsc.VectorSubcoreMesh(core_axis_name="c", subcore_axis_name="s",
                                    num_cores=_NC, num_subcores=_NS),
        compiler_params=pltpu.CompilerParams(use_tc_tiling_on_sc=False),
        scratch_types=(
            [pltpu.VMEM((_BPW,), jnp.int32) for _ in range(4)]
            + [pltpu.VMEM((_BPC, 4 * EMB), jnp.float32) for _ in range(4)]
            + [pltpu.SemaphoreType.DMA]
        ),
    )


_BBLK = 1024


def _quarter(g, q):
    out = jnp.where(q == 0, g[:, :EMB], 0.0)
    for k in range(1, 4):
        out = out + jnp.where(q == k, g[:, k * EMB:(k + 1) * EMB], 0.0)
    return out


def _tc_mlp_body(g0_ref, g1_ref, g2_ref, g3_ref, q_ref, num_ref,
                 ge_ref, be_ref, gn_ref, bn_ref,
                 w1a_ref, w1b_ref, b1_ref, g1l_ref, bb1_ref,
                 w2_ref, b2_ref, o_ref):
    q = q_ref[...]
    e = jnp.concatenate(
        [_quarter(g0_ref[...], q[:, 0:1]),
         _quarter(g1_ref[...], q[:, 1:2]),
         _quarter(g2_ref[...], q[:, 2:3]),
         _quarter(g3_ref[...], q[:, 3:4])], axis=-1)
    num = num_ref[...]
    inv_n = 1.0 / N_FEAT
    s = jnp.sum(e, axis=-1, keepdims=True) + jnp.sum(num, axis=-1, keepdims=True)
    mu = s * inv_n
    ss = (jnp.sum(e * e, axis=-1, keepdims=True)
          + jnp.sum(num * num, axis=-1, keepdims=True))
    var = ss * inv_n - mu * mu
    rstd = lax.rsqrt(var + 1e-5)
    en = (e - mu) * rstd * ge_ref[...] + be_ref[...]
    nn = (num - mu) * rstd * gn_ref[...] + bn_ref[...]
    h = (jnp.dot(en, w1a_ref[...], preferred_element_type=jnp.float32)
         + jnp.dot(nn, w1b_ref[...], preferred_element_type=jnp.float32)
         + b1_ref[...])
    h = jnp.maximum(h, 0.0)
    mu1 = jnp.mean(h, axis=-1, keepdims=True)
    var1 = jnp.mean(h * h, axis=-1, keepdims=True) - mu1 * mu1
    hn = (h - mu1) * lax.rsqrt(var1 + 1e-5) * g1l_ref[...] + bb1_ref[...]
    o = jnp.dot(hn, w2_ref[...], preferred_element_type=jnp.float32) + b2_ref[...]
    nrm = jnp.maximum(jnp.sqrt(jnp.sum(o * o, axis=-1, keepdims=True)), 1e-8)
    o_ref[...] = o / nrm


def _full(shape):
    return pl.BlockSpec(shape, lambda i: (0,) * len(shape))


_tc_mlp = pl.pallas_call(
    _tc_mlp_body,
    grid=(B // _BBLK,),
    in_specs=(
        [pl.BlockSpec((_BBLK, 4 * EMB), lambda i: (i, 0)) for _ in range(4)]
        + [
            pl.BlockSpec((_BBLK, 4), lambda i: (i, 0)),
            pl.BlockSpec((_BBLK, NUM), lambda i: (i, 0)),
            _full((1, 4 * EMB)), _full((1, 4 * EMB)),
            _full((1, NUM)), _full((1, NUM)),
            _full((4 * EMB, HID)), _full((NUM, HID)), _full((1, HID)),
            _full((1, HID)), _full((1, HID)),
            _full((HID, OUT)), _full((1, OUT)),
        ]
    ),
    out_specs=pl.BlockSpec((_BBLK, OUT), lambda i: (i, 0)),
    out_shape=jax.ShapeDtypeStruct((B, OUT), jnp.float32),
)


@jax.jit
def kernel(pt_enc, ig_enc, cg_enc, gg_enc, item_num, pt_tab, ig_tab, cg_tab,
           gg_tab, ln0_g, ln0_b, W1, b1, ln1_g, ln1_b, W2, b2):
    encs = [e.astype(jnp.int32) for e in (pt_enc, ig_enc, cg_enc, gg_enc)]
    qs = [e // _Q for e in encs]
    fused = [e - qv * _Q for e, qv in zip(encs, qs)]
    q = jnp.stack(qs, axis=-1)
    tts = [pt_tab.T, ig_tab.T, cg_tab.T, gg_tab.T]
    p0, p1, p2, p3 = _tc_transpose(*[tt for tt in tts for _ in range(4)])
    g0, g1, g2, g3 = _sc_gather()(*fused, p0, p1, p2, p3)
    ge = ln0_g[:4 * EMB].reshape(1, -1)
    be = ln0_b[:4 * EMB].reshape(1, -1)
    gn = ln0_g[4 * EMB:].reshape(1, -1)
    bn = ln0_b[4 * EMB:].reshape(1, -1)
    return _tc_mlp(g0, g1, g2, g3, q, item_num, ge, be, gn, bn,
                   W1[:4 * EMB], W1[4 * EMB:], b1.reshape(1, -1),
                   ln1_g.reshape(1, -1), ln1_b.reshape(1, -1),
                   W2, b2.reshape(1, -1))


# speedup vs baseline: 1.3851x; 1.3851x over previous
import functools

import jax
import jax.numpy as jnp
from jax import lax
from jax.experimental import pallas as pl
from jax.experimental.pallas import tpu as pltpu
from jax.experimental.pallas import tpu_sc as plsc

B = 16384
V1 = 100001
EMB = 32
HID = 256
OUT = 128
NUM = 3
N_FEAT = 131

_TBLK = 1024
_NTB = 98
_VPAD = _TBLK * _NTB

_NC, _NS = 2, 16
_NW = _NC * _NS
_BPW = B // _NW
_BPC = 128


_Q = _VPAD // 4


_QB = _TBLK // 4


def _transpose_body(*refs):
    xs, outs = refs[:16], refs[16:]
    ident = (lax.broadcasted_iota(jnp.int32, (4 * EMB, 4 * EMB), 0)
             == lax.broadcasted_iota(jnp.int32, (4 * EMB, 4 * EMB), 1)
             ).astype(jnp.float32)
    for t in range(4):
        x4 = jnp.concatenate([xs[4 * t + a][...] for a in range(4)], axis=0)
        outs[t][...] = lax.dot_general(
            x4, ident, (((0,), (0,)), ((), ())),
            preferred_element_type=jnp.float32)


def _tquarter_spec(a):
    return pl.BlockSpec((EMB, _QB), lambda i, a=a: (0, a * _NTB + i))


_tc_transpose = pl.pallas_call(
    _transpose_body,
    grid=(_NTB,),
    in_specs=[_tquarter_spec(a) for _ in range(4) for a in range(4)],
    out_specs=[pl.BlockSpec((_QB, 4 * EMB), lambda i: (i, 0))
               for _ in range(4)],
    out_shape=[jax.ShapeDtypeStruct((_Q, 4 * EMB), jnp.float32)] * 4,
)


def _sc_gather_body(i0, i1, i2, i3, t0, t1, t2, t3, o0, o1, o2, o3,
                    idx0, idx1, idx2, idx3, r0, r1, r2, r3, sem):
    wid = lax.axis_index("s") * _NC + lax.axis_index("c")
    base = wid * _BPW
    idxs = (idx0, idx1, idx2, idx3)
    rows = (r0, r1, r2, r3)
    tabs = (t0, t1, t2, t3)
    encs = (i0, i1, i2, i3)
    outs = (o0, o1, o2, o3)
    for t in range(4):
        pltpu.sync_copy(encs[t].at[pl.ds(base, _BPW)], idxs[t])
    for c in range(_BPW // _BPC):
        cbase = c * _BPC
        cps = [pltpu.async_copy(tabs[t].at[idxs[t].at[pl.ds(cbase, _BPC)]],
                                rows[t], sem)
               for t in range(4)]
        for cp in cps:
            cp.wait()
        for t in range(4):
            pltpu.sync_copy(rows[t],
                            outs[t].at[pl.ds(base + cbase, _BPC), :])


@functools.cache
def _sc_gather():
    return pl.kernel(
        _sc_gather_body,
        out_type=[jax.ShapeDtypeStruct((B, 4 * EMB), jnp.float32)] * 4,
        mesh=plsc.VectorSubcoreMesh(core_axis_name="c", subcore_axis_name="s",
                                    num_cores=_NC, num_subcores=_NS),
        compiler_params=pltpu.CompilerParams(use_tc_tiling_on_sc=False),
        scratch_types=(
            [pltpu.VMEM((_BPW,), jnp.int32) for _ in range(4)]
            + [pltpu.VMEM((_BPC, 4 * EMB), jnp.float32) for _ in range(4)]
            + [pltpu.SemaphoreType.DMA]
        ),
    )


_BBLK = 1024


def _quarter(g, q):
    out = jnp.where(q == 0, g[:, :EMB], 0.0)
    for k in range(1, 4):
        out = out + jnp.where(q == k, g[:, k * EMB:(k + 1) * EMB], 0.0)
    return out


def _tc_mlp_body(g0_ref, g1_ref, g2_ref, g3_ref, q_ref, num_ref,
                 ge_ref, be_ref, gn_ref, bn_ref,
                 w1a_ref, w1b_ref, b1_ref, g1l_ref, bb1_ref,
                 w2_ref, b2_ref, o_ref):
    q = q_ref[...]
    e = jnp.concatenate(
        [_quarter(g0_ref[...], q[:, 0:1]),
         _quarter(g1_ref[...], q[:, 1:2]),
         _quarter(g2_ref[...], q[:, 2:3]),
         _quarter(g3_ref[...], q[:, 3:4])], axis=-1)
    num = num_ref[...]
    inv_n = 1.0 / N_FEAT
    s = jnp.sum(e, axis=-1, keepdims=True) + jnp.sum(num, axis=-1, keepdims=True)
    mu = s * inv_n
    ss = (jnp.sum(e * e, axis=-1, keepdims=True)
          + jnp.sum(num * num, axis=-1, keepdims=True))
    var = ss * inv_n - mu * mu
    rstd = lax.rsqrt(var + 1e-5)
    en = (e - mu) * rstd * ge_ref[...] + be_ref[...]
    nn = (num - mu) * rstd * gn_ref[...] + bn_ref[...]
    h = (jnp.dot(en, w1a_ref[...], preferred_element_type=jnp.float32)
         + jnp.dot(nn, w1b_ref[...], preferred_element_type=jnp.float32)
         + b1_ref[...])
    h = jnp.maximum(h, 0.0)
    mu1 = jnp.mean(h, axis=-1, keepdims=True)
    var1 = jnp.mean(h * h, axis=-1, keepdims=True) - mu1 * mu1
    hn = (h - mu1) * lax.rsqrt(var1 + 1e-5) * g1l_ref[...] + bb1_ref[...]
    o = jnp.dot(hn, w2_ref[...], preferred_element_type=jnp.float32) + b2_ref[...]
    nrm = jnp.maximum(jnp.sqrt(jnp.sum(o * o, axis=-1, keepdims=True)), 1e-8)
    o_ref[...] = o / nrm


def _full(shape):
    return pl.BlockSpec(shape, lambda i: (0,) * len(shape))


_tc_mlp = pl.pallas_call(
    _tc_mlp_body,
    grid=(B // _BBLK,),
    in_specs=(
        [pl.BlockSpec((_BBLK, 4 * EMB), lambda i: (i, 0)) for _ in range(4)]
        + [
            pl.BlockSpec((_BBLK, 4), lambda i: (i, 0)),
            pl.BlockSpec((_BBLK, NUM), lambda i: (i, 0)),
            _full((1, 4 * EMB)), _full((1, 4 * EMB)),
            _full((1, NUM)), _full((1, NUM)),
            _full((4 * EMB, HID)), _full((NUM, HID)), _full((1, HID)),
            _full((1, HID)), _full((1, HID)),
            _full((HID, OUT)), _full((1, OUT)),
        ]
    ),
    out_specs=pl.BlockSpec((_BBLK, OUT), lambda i: (i, 0)),
    out_shape=jax.ShapeDtypeStruct((B, OUT), jnp.float32),
)


@jax.jit
def kernel(pt_enc, ig_enc, cg_enc, gg_enc, item_num, pt_tab, ig_tab, cg_tab,
           gg_tab, ln0_g, ln0_b, W1, b1, ln1_g, ln1_b, W2, b2):
    encs = [e.astype(jnp.int32) for e in (pt_enc, ig_enc, cg_enc, gg_enc)]
    qs = [e // _Q for e in encs]
    fused = [e - qv * _Q for e, qv in zip(encs, qs)]
    q = jnp.stack(qs, axis=-1)
    tts = [pt_tab.T, ig_tab.T, cg_tab.T, gg_tab.T]
    p0, p1, p2, p3 = _tc_transpose(*[tt for tt in tts for _ in range(4)])
    g0, g1, g2, g3 = _sc_gather()(*fused, p0, p1, p2, p3)
    ge = ln0_g[:4 * EMB].reshape(1, -1)
    be = ln0_b[:4 * EMB].reshape(1, -1)
    gn = ln0_g[4 * EMB:].reshape(1, -1)
    bn = ln0_b[4 * EMB:].reshape(1, -1)
    return _tc_mlp(g0, g1, g2, g3, q, item_num, ge, be, gn, bn,
                   W1[:4 * EMB], W1[4 * EMB:], b1.reshape(1, -1),
                   ln1_g.reshape(1, -1), ln1_b.reshape(1, -1),
                   W2, b2.reshape(1, -1))


# speedup vs baseline: 1.6247x; 1.1730x over previous
import functools

import jax
import jax.numpy as jnp
from jax import lax
from jax.experimental import pallas as pl
from jax.experimental.pallas import tpu as pltpu
from jax.experimental.pallas import tpu_sc as plsc

B = 16384
V1 = 100001
EMB = 32
HID = 256
OUT = 128
NUM = 3
N_FEAT = 131

_TBLK = 1024
_NTB = 98
_VPAD = _TBLK * _NTB

_NC, _NS = 2, 16
_NW = _NC * _NS
_BPW = B // _NW
_BPC = 128


_Q = _VPAD // 4


_QB = _TBLK // 4


def _transpose_body(*refs):
    xs, outs = refs[:16], refs[16:]
    ident = (lax.broadcasted_iota(jnp.int32, (4 * EMB, 4 * EMB), 0)
             == lax.broadcasted_iota(jnp.int32, (4 * EMB, 4 * EMB), 1)
             ).astype(jnp.float32)
    for t in range(4):
        x4 = jnp.concatenate([xs[4 * t + a][...] for a in range(4)], axis=0)
        outs[t][...] = lax.dot_general(
            x4, ident, (((0,), (0,)), ((), ())),
            preferred_element_type=jnp.float32)


def _tquarter_spec(a):
    return pl.BlockSpec((EMB, _QB), lambda i, a=a: (0, a * _NTB + i))


_tc_transpose = pl.pallas_call(
    _transpose_body,
    grid=(_NTB,),
    in_specs=[_tquarter_spec(a) for _ in range(4) for a in range(4)],
    out_specs=[pl.BlockSpec((_QB, 4 * EMB), lambda i: (i, 0))
               for _ in range(4)],
    out_shape=[jax.ShapeDtypeStruct((_Q, 4 * EMB), jnp.float32)] * 4,
)


def _sc_gather_body(i0, i1, i2, i3, t0, t1, t2, t3, o0, o1, o2, o3,
                    idx0, idx1, idx2, idx3, r0, r1, r2, r3, sem):
    wid = lax.axis_index("s") * _NC + lax.axis_index("c")
    base = wid * _BPW
    idxs = (idx0, idx1, idx2, idx3)
    rows = (r0, r1, r2, r3)
    tabs = (t0, t1, t2, t3)
    encs = (i0, i1, i2, i3)
    outs = (o0, o1, o2, o3)
    for t in range(4):
        pltpu.sync_copy(encs[t].at[pl.ds(base, _BPW)], idxs[t])
    for c in range(_BPW // _BPC):
        cbase = c * _BPC
        cps = [pltpu.async_copy(tabs[t].at[idxs[t].at[pl.ds(cbase, _BPC)]],
                                rows[t], sem)
               for t in range(4)]
        for cp in cps:
            cp.wait()
        for t in range(4):
            pltpu.sync_copy(rows[t],
                            outs[t].at[pl.ds(base + cbase, _BPC), :])


@functools.cache
def _sc_gather():
    return pl.kernel(
        _sc_gather_body,
        out_type=[jax.ShapeDtypeStruct((B, 4 * EMB), jnp.float32)] * 4,
        mesh=plsc.VectorSubcoreMesh(core_axis_name="c", subcore_axis_name="s",
                                    num_cores=_NC, num_subcores=_NS),
        compiler_params=pltpu.CompilerParams(use_tc_tiling_on_sc=False),
        scratch_types=(
            [pltpu.VMEM((_BPW,), jnp.int32) for _ in range(4)]
            + [pltpu.VMEM((_BPC, 4 * EMB), jnp.float32) for _ in range(4)]
            + [pltpu.SemaphoreType.DMA]
        ),
    )


_BBLK = 1024


def _quarter(g, q):
    lane_q = lax.broadcasted_iota(jnp.int32, g.shape, 1) // EMB
    m = jnp.where(lane_q == q, g, 0.0)
    return ((m[:, :EMB] + m[:, EMB:2 * EMB])
            + (m[:, 2 * EMB:3 * EMB] + m[:, 3 * EMB:]))


def _tc_mlp_body(g0_ref, g1_ref, g2_ref, g3_ref, q_ref, num_ref,
                 ge_ref, be_ref, gn_ref, bn_ref,
                 w1a_ref, w1b_ref, b1_ref, g1l_ref, bb1_ref,
                 w2_ref, b2_ref, o_ref):
    q = q_ref[...]
    e = jnp.concatenate(
        [_quarter(g0_ref[...], q[:, 0:1]),
         _quarter(g1_ref[...], q[:, 1:2]),
         _quarter(g2_ref[...], q[:, 2:3]),
         _quarter(g3_ref[...], q[:, 3:4])], axis=-1)
    num = num_ref[...]
    inv_n = 1.0 / N_FEAT
    s = jnp.sum(e, axis=-1, keepdims=True) + jnp.sum(num, axis=-1, keepdims=True)
    mu = s * inv_n
    ss = (jnp.sum(e * e, axis=-1, keepdims=True)
          + jnp.sum(num * num, axis=-1, keepdims=True))
    var = ss * inv_n - mu * mu
    rstd = lax.rsqrt(var + 1e-5)
    en = (e - mu) * rstd * ge_ref[...] + be_ref[...]
    nn = (num - mu) * rstd * gn_ref[...] + bn_ref[...]
    h = (jnp.dot(en, w1a_ref[...], preferred_element_type=jnp.float32)
         + jnp.dot(nn, w1b_ref[...], preferred_element_type=jnp.float32)
         + b1_ref[...])
    h = jnp.maximum(h, 0.0)
    mu1 = jnp.mean(h, axis=-1, keepdims=True)
    var1 = jnp.mean(h * h, axis=-1, keepdims=True) - mu1 * mu1
    hn = (h - mu1) * lax.rsqrt(var1 + 1e-5) * g1l_ref[...] + bb1_ref[...]
    o = jnp.dot(hn, w2_ref[...], preferred_element_type=jnp.float32) + b2_ref[...]
    nrm = jnp.maximum(jnp.sqrt(jnp.sum(o * o, axis=-1, keepdims=True)), 1e-8)
    o_ref[...] = o / nrm


def _full(shape):
    return pl.BlockSpec(shape, lambda i: (0,) * len(shape))


_tc_mlp = pl.pallas_call(
    _tc_mlp_body,
    grid=(B // _BBLK,),
    in_specs=(
        [pl.BlockSpec((_BBLK, 4 * EMB), lambda i: (i, 0)) for _ in range(4)]
        + [
            pl.BlockSpec((_BBLK, 4), lambda i: (i, 0)),
            pl.BlockSpec((_BBLK, NUM), lambda i: (i, 0)),
            _full((1, 4 * EMB)), _full((1, 4 * EMB)),
            _full((1, NUM)), _full((1, NUM)),
            _full((4 * EMB, HID)), _full((NUM, HID)), _full((1, HID)),
            _full((1, HID)), _full((1, HID)),
            _full((HID, OUT)), _full((1, OUT)),
        ]
    ),
    out_specs=pl.BlockSpec((_BBLK, OUT), lambda i: (i, 0)),
    out_shape=jax.ShapeDtypeStruct((B, OUT), jnp.float32),
)


@jax.jit
def kernel(pt_enc, ig_enc, cg_enc, gg_enc, item_num, pt_tab, ig_tab, cg_tab,
           gg_tab, ln0_g, ln0_b, W1, b1, ln1_g, ln1_b, W2, b2):
    encs = [e.astype(jnp.int32) for e in (pt_enc, ig_enc, cg_enc, gg_enc)]
    qs = [e // _Q for e in encs]
    fused = [e - qv * _Q for e, qv in zip(encs, qs)]
    q = jnp.stack(qs, axis=-1)
    tts = [pt_tab.T, ig_tab.T, cg_tab.T, gg_tab.T]
    p0, p1, p2, p3 = _tc_transpose(*[tt for tt in tts for _ in range(4)])
    g0, g1, g2, g3 = _sc_gather()(*fused, p0, p1, p2, p3)
    ge = ln0_g[:4 * EMB].reshape(1, -1)
    be = ln0_b[:4 * EMB].reshape(1, -1)
    gn = ln0_g[4 * EMB:].reshape(1, -1)
    bn = ln0_b[4 * EMB:].reshape(1, -1)
    return _tc_mlp(g0, g1, g2, g3, q, item_num, ge, be, gn, bn,
                   W1[:4 * EMB], W1[4 * EMB:], b1.reshape(1, -1),
                   ln1_g.reshape(1, -1), ln1_b.reshape(1, -1),
                   W2, b2.reshape(1, -1))
